# baseline (device time: 78469 ns/iter reference)
import jax
import jax.numpy as jnp
from jax import lax
from jax.experimental import pallas as pl
from jax.experimental.pallas import tpu as pltpu

N_DEV = 8
W = 2


def kernel(x):
    m_per, n = x.shape
    n_per = n // N_DEV

    def body(x_ref, out_ref, buf_ref, send_sems, recv_sems):
        my = lax.axis_index("i")
        my_code = my ^ ((my >> 1) & 1)

        def partner(t):
            c = my_code ^ t
            return c ^ ((c >> 1) & 1)

        barrier = pltpu.get_barrier_semaphore()
        for k in range(1, N_DEV):
            pl.semaphore_signal(
                barrier, inc=1,
                device_id=((my + k) % N_DEV,),
                device_id_type=pl.DeviceIdType.MESH,
            )
        pl.semaphore_wait(barrier, N_DEV - 1)

        def start_send(t):
            tgt = partner(t)
            col = tgt * n_per
            buf_ref[t - 1, :, :] = (
                x_ref[:, pl.ds(col, n_per)].astype(jnp.bfloat16)
            )
            rdma = pltpu.make_async_remote_copy(
                src_ref=buf_ref.at[t - 1],
                dst_ref=out_ref.at[pl.ds(my * m_per, m_per), :],
                send_sem=send_sems.at[t - 1],
                recv_sem=recv_sems.at[t - 1],
                device_id=(tgt,),
                device_id_type=pl.DeviceIdType.MESH,
            )
            rdma.start()
            return rdma

        def wait_recv(t):
            src = partner(t)
            recv = pltpu.make_async_remote_copy(
                src_ref=buf_ref.at[0],
                dst_ref=out_ref.at[pl.ds(src * m_per, m_per), :],
                send_sem=send_sems.at[t - 1],
                recv_sem=recv_sems.at[t - 1],
                device_id=(src,),
                device_id_type=pl.DeviceIdType.MESH,
            )
            recv.wait_recv()

        sends = []
        for t in range(1, W + 1):
            sends.append(start_send(t))

        out_ref[pl.ds(my * m_per, m_per), :] = (
            x_ref[:, pl.ds(my * n_per, n_per)].astype(jnp.bfloat16)
        )

        for t in range(W + 1, N_DEV):
            wait_recv(t - W)
            sends.append(start_send(t))

        for t in range(N_DEV - W, N_DEV):
            wait_recv(t)

        for rdma in sends:
            rdma.wait_send()

    return pl.pallas_call(
        body,
        out_shape=jax.ShapeDtypeStruct((N_DEV * m_per, n_per), jnp.bfloat16),
        in_specs=[pl.BlockSpec(memory_space=pltpu.VMEM)],
        out_specs=pl.BlockSpec(memory_space=pltpu.VMEM),
        scratch_shapes=[
            pltpu.VMEM((N_DEV - 1, m_per, n_per), jnp.bfloat16),
            pltpu.SemaphoreType.DMA((N_DEV - 1,)),
            pltpu.SemaphoreType.DMA((N_DEV - 1,)),
        ],
        compiler_params=pltpu.CompilerParams(collective_id=0),
    )(x)


# device time: 61228 ns/iter; 1.2816x vs baseline; 1.2816x over previous
import jax
import jax.numpy as jnp
from jax import lax
from jax.experimental import pallas as pl
from jax.experimental.pallas import tpu as pltpu

N_DEV = 8
STAGE_ORDER = (1, 6, 2, 5, 4, 3, 7)


def kernel(x):
    m_per, n = x.shape
    n_per = n // N_DEV

    def body(x_ref, out_ref, buf_ref, send_sems, recv_sems):
        my = lax.axis_index("i")
        my_code = my ^ ((my >> 1) & 1)

        def partner(t):
            c = my_code ^ t
            return c ^ ((c >> 1) & 1)

        barrier = pltpu.get_barrier_semaphore()
        for k in range(1, N_DEV):
            pl.semaphore_signal(
                barrier, inc=1,
                device_id=((my + k) % N_DEV,),
                device_id_type=pl.DeviceIdType.MESH,
            )
        pl.semaphore_wait(barrier, N_DEV - 1)

        sends = []
        for si, t in enumerate(STAGE_ORDER):
            tgt = partner(t)
            buf_ref[si, :, :] = (
                x_ref[:, pl.ds(tgt * n_per, n_per)].astype(jnp.bfloat16)
            )
            rdma = pltpu.make_async_remote_copy(
                src_ref=buf_ref.at[si],
                dst_ref=out_ref.at[pl.ds(my * m_per, m_per), :],
                send_sem=send_sems.at[si],
                recv_sem=recv_sems.at[si],
                device_id=(tgt,),
                device_id_type=pl.DeviceIdType.MESH,
            )
            rdma.start()
            sends.append(rdma)

        out_ref[pl.ds(my * m_per, m_per), :] = (
            x_ref[:, pl.ds(my * n_per, n_per)].astype(jnp.bfloat16)
        )

        for si, t in enumerate(STAGE_ORDER):
            src = partner(t)
            recv = pltpu.make_async_remote_copy(
                src_ref=buf_ref.at[0],
                dst_ref=out_ref.at[pl.ds(src * m_per, m_per), :],
                send_sem=send_sems.at[si],
                recv_sem=recv_sems.at[si],
                device_id=(src,),
                device_id_type=pl.DeviceIdType.MESH,
            )
            recv.wait_recv()

        for rdma in sends:
            rdma.wait_send()

    return pl.pallas_call(
        body,
        out_shape=jax.ShapeDtypeStruct((N_DEV * m_per, n_per), jnp.bfloat16),
        in_specs=[pl.BlockSpec(memory_space=pltpu.VMEM)],
        out_specs=pl.BlockSpec(memory_space=pltpu.VMEM),
        scratch_shapes=[
            pltpu.VMEM((N_DEV - 1, m_per, n_per), jnp.bfloat16),
            pltpu.SemaphoreType.DMA((N_DEV - 1,)),
            pltpu.SemaphoreType.DMA((N_DEV - 1,)),
        ],
        compiler_params=pltpu.CompilerParams(collective_id=0),
    )(x)


# device time: 55998 ns/iter; 1.4013x vs baseline; 1.0934x over previous
import jax
import jax.numpy as jnp
from jax import lax
from jax.experimental import pallas as pl
from jax.experimental.pallas import tpu as pltpu

N_DEV = 8
STAGE_ORDER = (1, 6, 2, 5, 4, 3, 7)


def kernel(x):
    m_per, n = x.shape
    n_per = n // N_DEV

    def body(x_ref, out_ref, xv_ref, buf_ref, copy_sems, send_sems, recv_sems):
        my = lax.axis_index("i")
        my_code = my ^ ((my >> 1) & 1)

        def partner(t):
            c = my_code ^ t
            return c ^ ((c >> 1) & 1)

        copies = []
        for si, t in enumerate(STAGE_ORDER):
            tgt = partner(t)
            cp = pltpu.make_async_copy(
                x_ref.at[:, pl.ds(tgt * n_per, n_per)],
                xv_ref.at[si],
                copy_sems.at[si],
            )
            cp.start()
            copies.append(cp)
        own_cp = pltpu.make_async_copy(
            x_ref.at[:, pl.ds(my * n_per, n_per)],
            xv_ref.at[N_DEV - 1],
            copy_sems.at[N_DEV - 1],
        )
        own_cp.start()

        barrier = pltpu.get_barrier_semaphore()
        for k in range(1, N_DEV):
            pl.semaphore_signal(
                barrier, inc=1,
                device_id=((my + k) % N_DEV,),
                device_id_type=pl.DeviceIdType.MESH,
            )
        pl.semaphore_wait(barrier, N_DEV - 1)

        sends = []
        for si, t in enumerate(STAGE_ORDER):
            tgt = partner(t)
            copies[si].wait()
            buf_ref[si, :, :] = xv_ref[si].astype(jnp.bfloat16)
            rdma = pltpu.make_async_remote_copy(
                src_ref=buf_ref.at[si],
                dst_ref=out_ref.at[pl.ds(my * m_per, m_per), :],
                send_sem=send_sems.at[si],
                recv_sem=recv_sems.at[si],
                device_id=(tgt,),
                device_id_type=pl.DeviceIdType.MESH,
            )
            rdma.start()
            sends.append(rdma)

        own_cp.wait()
        out_ref[pl.ds(my * m_per, m_per), :] = (
            xv_ref[N_DEV - 1].astype(jnp.bfloat16)
        )

        for si, t in enumerate(STAGE_ORDER):
            src = partner(t)
            recv = pltpu.make_async_remote_copy(
                src_ref=buf_ref.at[0],
                dst_ref=out_ref.at[pl.ds(src * m_per, m_per), :],
                send_sem=send_sems.at[si],
                recv_sem=recv_sems.at[si],
                device_id=(src,),
                device_id_type=pl.DeviceIdType.MESH,
            )
            recv.wait_recv()

        for rdma in sends:
            rdma.wait_send()

    return pl.pallas_call(
        body,
        out_shape=jax.ShapeDtypeStruct((N_DEV * m_per, n_per), jnp.bfloat16),
        in_specs=[pl.BlockSpec(memory_space=pl.ANY)],
        out_specs=pl.BlockSpec(memory_space=pltpu.VMEM),
        scratch_shapes=[
            pltpu.VMEM((N_DEV, m_per, n_per), x.dtype),
            pltpu.VMEM((N_DEV - 1, m_per, n_per), jnp.bfloat16),
            pltpu.SemaphoreType.DMA((N_DEV,)),
            pltpu.SemaphoreType.DMA((N_DEV - 1,)),
            pltpu.SemaphoreType.DMA((N_DEV - 1,)),
        ],
        compiler_params=pltpu.CompilerParams(collective_id=0),
    )(x)
